# baseline (speedup 1.0000x reference)
import functools

import jax
import jax.numpy as jnp
from jax import lax
from jax.experimental import pallas as pl
from jax.experimental.pallas import tpu as pltpu
from jax.experimental.pallas import tpu_sc as plsc

TW = 5
N_NEIGH = 16
NT = 250
TMIN = 0.0
TMAX = 1.0

NX = 4096
ROW_TILE = 256
PAD = 16
NPAD = NX + 2 * PAD

NC = 2
NS = 16
NW = NC * NS
NODES_PER_W = NX // NW
LANES = 16

_BIG_X = 3e37
_BIG_I = 1 << 22


def _rank_body(xr_ref, xc_ref, data_ref, labels_ref, out_ref, u_ref, y_ref):
    i = pl.program_id(0)
    xi = jnp.transpose(xc_ref[...], (1, 0))
    xj = xr_ref[...]
    lt = xj < xi
    eq = xj == xi
    jj = lax.broadcasted_iota(jnp.int32, (ROW_TILE, NX), 1)
    ii = lax.broadcasted_iota(jnp.int32, (ROW_TILE, NX), 0) + i * ROW_TILE
    before = lt | (eq & (jj < ii))
    out_ref[...] = jnp.sum(before.astype(jnp.int32), axis=1, keepdims=True)
    u_ref[...] = jnp.transpose(data_ref[0], (1, 0))
    y_ref[...] = jnp.transpose(labels_ref[0], (1, 0))


def _ranks_uy_tc(xrow, data, labels):
    B = data.shape[0]
    ranks, u, y = pl.pallas_call(
        _rank_body,
        grid=(NX // ROW_TILE,),
        in_specs=[
            pl.BlockSpec((1, NX), lambda i: (0, 0)),
            pl.BlockSpec((1, ROW_TILE), lambda i: (0, i)),
            pl.BlockSpec((1, TW, NX), lambda i: (i, 0, 0)),
            pl.BlockSpec((1, TW, NX), lambda i: (i, 0, 0)),
        ],
        out_specs=[
            pl.BlockSpec((ROW_TILE, 1), lambda i: (i, 0)),
            pl.BlockSpec((NX, TW), lambda i: (i, 0)),
            pl.BlockSpec((NX, TW), lambda i: (i, 0)),
        ],
        out_shape=[
            jax.ShapeDtypeStruct((NX, 1), jnp.int32),
            jax.ShapeDtypeStruct((B * NX, TW), jnp.float32),
            jax.ShapeDtypeStruct((B * NX, TW), jnp.float32),
        ],
    )(xrow, xrow, data, labels)
    return ranks.reshape(NX), u, y


def _sc_knn_body(x_hbm, r_hbm, out_hbm, vx, vr, sx, si, vout):
    cid = lax.axis_index("c")
    sid = lax.axis_index("s")
    wid = cid * NS + sid

    pltpu.sync_copy(x_hbm, vx)
    pltpu.sync_copy(r_hbm, vr)

    big_x = jnp.full((LANES,), _BIG_X, jnp.float32)
    big_i = jnp.full((LANES,), _BIG_I, jnp.int32)
    sx[pl.ds(0, LANES)] = big_x
    sx[pl.ds(NX + PAD, LANES)] = big_x
    si[pl.ds(0, LANES)] = big_i
    si[pl.ds(NX + PAD, LANES)] = big_i

    iota = lax.iota(jnp.int32, LANES)

    def scat(cidx, carry):
        base = cidx * LANES
        rv = vr[pl.ds(base, LANES)] + PAD
        plsc.store_scatter(sx, [rv], vx[pl.ds(base, LANES)])
        plsc.store_scatter(si, [rv], iota + base)
        return carry

    lax.fori_loop(0, NX // LANES, scat, 0)

    def node_vec(v, carry):
        base = wid * NODES_PER_W + v * LANES
        r = vr[pl.ds(base, LANES)] + PAD
        xi = vx[pl.ds(base, LANES)]
        rowi = iota + v * LANES
        lp = jnp.ones((LANES,), jnp.int32)
        rp = jnp.ones((LANES,), jnp.int32)
        for step in range(N_NEIGH):
            gl = r - lp
            gr = r + rp
            xl = plsc.load_gather(sx, [gl])
            il = plsc.load_gather(si, [gl])
            xr_ = plsc.load_gather(sx, [gr])
            ir = plsc.load_gather(si, [gr])
            dl = jnp.abs(xi - xl)
            dr = jnp.abs(xi - xr_)
            take_l = (dl < dr) | ((dl == dr) & (il < ir))
            sel = jnp.where(take_l, il, ir)
            col = jnp.full((LANES,), step, jnp.int32)
            plsc.store_scatter(vout, [rowi, col], sel)
            inc = take_l.astype(jnp.int32)
            lp = lp + inc
            rp = rp + (1 - inc)
        return carry

    lax.fori_loop(0, NODES_PER_W // LANES, node_vec, 0)

    pltpu.sync_copy(vout, out_hbm.at[pl.ds(wid * NODES_PER_W, NODES_PER_W)])


def _knn_sc(x0, ranks):
    mesh = plsc.VectorSubcoreMesh(
        core_axis_name="c", subcore_axis_name="s",
        num_cores=NC, num_subcores=NS,
    )
    fn = pl.kernel(
        _sc_knn_body,
        out_type=jax.ShapeDtypeStruct((NX, N_NEIGH), jnp.int32),
        mesh=mesh,
        compiler_params=pltpu.CompilerParams(needs_layout_passes=False),
        scratch_types=[
            pltpu.VMEM((NX,), jnp.float32),
            pltpu.VMEM((NX,), jnp.int32),
            pltpu.VMEM((NPAD,), jnp.float32),
            pltpu.VMEM((NPAD,), jnp.int32),
            pltpu.VMEM((NODES_PER_W, N_NEIGH), jnp.int32),
        ],
    )
    return fn(x0, ranks)


def kernel(data, labels, x, steps, bc_left, bc_right, c):
    B, tw, nx = data.shape
    x0 = x[0]

    ranks, u, y = _ranks_uy_tc(x0.reshape(1, nx), data, labels)
    src0 = _knn_sc(x0, ranks)

    x_pos = jnp.broadcast_to(x0[None, :], (B, nx)).reshape(B * nx)
    t = jnp.linspace(TMIN, TMAX, NT)
    t_sel = t[steps]
    t_pos = jnp.broadcast_to(t_sel[:, None], (B, nx)).reshape(B * nx)
    batch = lax.broadcasted_iota(jnp.int32, (B, nx), 0).reshape(B * nx)

    offs = (jnp.arange(B, dtype=jnp.int32) * nx)[:, None]
    src = (src0.reshape(1, nx * N_NEIGH) + offs).reshape(1, B * nx * N_NEIGH)
    dst0 = lax.broadcasted_iota(jnp.int32, (nx, N_NEIGH), 0).reshape(1, nx * N_NEIGH)
    dst = (dst0 + offs).reshape(1, B * nx * N_NEIGH)
    edge_index = jnp.concatenate([src, dst], 0)

    pos = jnp.concatenate([t_pos[:, None], x_pos[:, None]], 1)
    bc_l = jnp.broadcast_to(bc_left[:, None], (B, nx)).reshape(B * nx, 1)
    bc_r = jnp.broadcast_to(bc_right[:, None], (B, nx)).reshape(B * nx, 1)
    c_n = jnp.broadcast_to(c[:, None], (B, nx)).reshape(B * nx, 1)
    return (u, edge_index, y, pos, batch, bc_l, bc_r, c_n)


# speedup vs baseline: 1.7141x; 1.7141x over previous
import functools

import jax
import jax.numpy as jnp
from jax import lax
from jax.experimental import pallas as pl
from jax.experimental.pallas import tpu as pltpu
from jax.experimental.pallas import tpu_sc as plsc

TW = 5
N_NEIGH = 16
NT = 250
TMIN = 0.0
TMAX = 1.0

NX = 4096
ROW_TILE = 256
PAD = 16
NPAD = NX + 2 * PAD

NC = 2
NS = 16
NW = NC * NS
NODES_PER_W = NX // NW
LANES = 16

_BIG_X = 3e37
_BIG_I = 1 << 22


def _rank_body(xr_ref, xc_ref, out_ref):
    i = pl.program_id(0)
    xi = jnp.transpose(xc_ref[...], (1, 0))
    xj = xr_ref[...]
    lt = xj < xi
    eq = xj == xi
    jj = lax.broadcasted_iota(jnp.int32, (ROW_TILE, NX), 1)
    ii = lax.broadcasted_iota(jnp.int32, (ROW_TILE, NX), 0) + i * ROW_TILE
    before = lt | (eq & (jj < ii))
    out_ref[...] = jnp.sum(before.astype(jnp.int32), axis=1, keepdims=True)


def _ranks_tc(xrow):
    ranks = pl.pallas_call(
        _rank_body,
        grid=(NX // ROW_TILE,),
        in_specs=[
            pl.BlockSpec((1, NX), lambda i: (0, 0)),
            pl.BlockSpec((1, ROW_TILE), lambda i: (0, i)),
        ],
        out_specs=pl.BlockSpec((ROW_TILE, 1), lambda i: (i, 0)),
        out_shape=jax.ShapeDtypeStruct((NX, 1), jnp.int32),
    )(xrow, xrow)
    return ranks.reshape(NX)


def _sc_knn_body(x_hbm, r_hbm, out_hbm, vx, vr, sx, si, vout):
    cid = lax.axis_index("c")
    sid = lax.axis_index("s")
    wid = cid * NS + sid

    pltpu.sync_copy(x_hbm, vx)
    pltpu.sync_copy(r_hbm, vr)

    big_x = jnp.full((LANES,), _BIG_X, jnp.float32)
    big_i = jnp.full((LANES,), _BIG_I, jnp.int32)
    sx[pl.ds(0, LANES)] = big_x
    sx[pl.ds(NX + PAD, LANES)] = big_x
    si[pl.ds(0, LANES)] = big_i
    si[pl.ds(NX + PAD, LANES)] = big_i

    iota = lax.iota(jnp.int32, LANES)

    def scat(cidx, carry):
        base = cidx * LANES
        rv = vr[pl.ds(base, LANES)] + PAD
        plsc.store_scatter(sx, [rv], vx[pl.ds(base, LANES)])
        plsc.store_scatter(si, [rv], iota + base)
        return carry

    lax.fori_loop(0, NX // LANES, scat, 0)

    def node_vec(v, carry):
        base = wid * NODES_PER_W + v * LANES
        r = vr[pl.ds(base, LANES)] + PAD
        xi = vx[pl.ds(base, LANES)]
        rowi = iota + v * LANES
        lp = jnp.ones((LANES,), jnp.int32)
        rp = jnp.ones((LANES,), jnp.int32)
        for step in range(N_NEIGH):
            gl = r - lp
            gr = r + rp
            xl = plsc.load_gather(sx, [gl])
            il = plsc.load_gather(si, [gl])
            xr_ = plsc.load_gather(sx, [gr])
            ir = plsc.load_gather(si, [gr])
            dl = jnp.abs(xi - xl)
            dr = jnp.abs(xi - xr_)
            take_l = (dl < dr) | ((dl == dr) & (il < ir))
            sel = jnp.where(take_l, il, ir)
            col = jnp.full((LANES,), step, jnp.int32)
            plsc.store_scatter(vout, [rowi, col], sel)
            inc = take_l.astype(jnp.int32)
            lp = lp + inc
            rp = rp + (1 - inc)
        return carry

    lax.fori_loop(0, NODES_PER_W // LANES, node_vec, 0)

    pltpu.sync_copy(vout, out_hbm.at[pl.ds(wid * NODES_PER_W, NODES_PER_W)])


def _knn_sc(x0, ranks):
    mesh = plsc.VectorSubcoreMesh(
        core_axis_name="c", subcore_axis_name="s",
        num_cores=NC, num_subcores=NS,
    )
    fn = pl.kernel(
        _sc_knn_body,
        out_type=jax.ShapeDtypeStruct((NX, N_NEIGH), jnp.int32),
        mesh=mesh,
        compiler_params=pltpu.CompilerParams(needs_layout_passes=False),
        scratch_types=[
            pltpu.VMEM((NX,), jnp.float32),
            pltpu.VMEM((NX,), jnp.int32),
            pltpu.VMEM((NPAD,), jnp.float32),
            pltpu.VMEM((NPAD,), jnp.int32),
            pltpu.VMEM((NODES_PER_W, N_NEIGH), jnp.int32),
        ],
    )
    return fn(x0, ranks)


def kernel(data, labels, x, steps, bc_left, bc_right, c):
    B, tw, nx = data.shape
    x0 = x[0]

    ranks = _ranks_tc(x0.reshape(1, nx))
    src0 = _knn_sc(x0, ranks)

    u = jnp.transpose(data, (0, 2, 1)).reshape(B * nx, tw)
    y = jnp.transpose(labels, (0, 2, 1)).reshape(B * nx, tw)

    x_pos = jnp.broadcast_to(x0[None, :], (B, nx)).reshape(B * nx)
    t = jnp.linspace(TMIN, TMAX, NT)
    t_sel = t[steps]
    t_pos = jnp.broadcast_to(t_sel[:, None], (B, nx)).reshape(B * nx)
    batch = lax.broadcasted_iota(jnp.int32, (B, nx), 0).reshape(B * nx)

    offs = (jnp.arange(B, dtype=jnp.int32) * nx)[:, None]
    src = (src0.reshape(1, nx * N_NEIGH) + offs).reshape(1, B * nx * N_NEIGH)
    dst0 = lax.broadcasted_iota(jnp.int32, (nx, N_NEIGH), 0).reshape(1, nx * N_NEIGH)
    dst = (dst0 + offs).reshape(1, B * nx * N_NEIGH)
    edge_index = jnp.concatenate([src, dst], 0)

    pos = jnp.concatenate([t_pos[:, None], x_pos[:, None]], 1)
    bc_l = jnp.broadcast_to(bc_left[:, None], (B, nx)).reshape(B * nx, 1)
    bc_r = jnp.broadcast_to(bc_right[:, None], (B, nx)).reshape(B * nx, 1)
    c_n = jnp.broadcast_to(c[:, None], (B, nx)).reshape(B * nx, 1)
    return (u, edge_index, y, pos, batch, bc_l, bc_r, c_n)


# speedup vs baseline: 1.7980x; 1.0489x over previous
import functools

import jax
import jax.numpy as jnp
from jax import lax
from jax.experimental import pallas as pl
from jax.experimental.pallas import tpu as pltpu
from jax.experimental.pallas import tpu_sc as plsc

TW = 5
N_NEIGH = 16
NT = 250
TMIN = 0.0
TMAX = 1.0

NX = 4096
ROW_TILE = 256
PAD = 16
NPAD = NX + 2 * PAD

NC = 2
NS = 16
NW = NC * NS
NODES_PER_W = NX // NW
LANES = 16

_BIG_X = 3e37
_BIG_I = 1 << 22


def _rank_body(xr_ref, xc_ref, out_ref):
    i = pl.program_id(0)
    ki = lax.bitcast_convert_type(
        jnp.transpose(xc_ref[...], (1, 0)), jnp.int32)
    kj = lax.bitcast_convert_type(xr_ref[...], jnp.int32)
    jj = lax.broadcasted_iota(jnp.int32, (ROW_TILE, NX), 1)
    ii = lax.broadcasted_iota(jnp.int32, (ROW_TILE, NX), 0) + i * ROW_TILE
    before = (kj - (jj < ii).astype(jnp.int32)) < ki
    out_ref[...] = jnp.sum(before.astype(jnp.int32), axis=1, keepdims=True)


def _ranks_tc(xrow):
    ranks = pl.pallas_call(
        _rank_body,
        grid=(NX // ROW_TILE,),
        in_specs=[
            pl.BlockSpec((1, NX), lambda i: (0, 0)),
            pl.BlockSpec((1, ROW_TILE), lambda i: (0, i)),
        ],
        out_specs=pl.BlockSpec((ROW_TILE, 1), lambda i: (i, 0)),
        out_shape=jax.ShapeDtypeStruct((NX, 1), jnp.int32),
    )(xrow, xrow)
    return ranks.reshape(NX)


def _sc_knn_body(x_hbm, r_hbm, out_hbm, vx, vr, sx, si, vout):
    cid = lax.axis_index("c")
    sid = lax.axis_index("s")
    wid = cid * NS + sid

    pltpu.sync_copy(x_hbm, vx)
    pltpu.sync_copy(r_hbm, vr)

    big_x = jnp.full((LANES,), _BIG_X, jnp.float32)
    big_i = jnp.full((LANES,), _BIG_I, jnp.int32)
    sx[pl.ds(0, LANES)] = big_x
    sx[pl.ds(NX + PAD, LANES)] = big_x
    si[pl.ds(0, LANES)] = big_i
    si[pl.ds(NX + PAD, LANES)] = big_i

    iota = lax.iota(jnp.int32, LANES)

    def scat(cidx, carry):
        base = cidx * LANES
        rv = vr[pl.ds(base, LANES)] + PAD
        plsc.store_scatter(sx, [rv], vx[pl.ds(base, LANES)])
        plsc.store_scatter(si, [rv], iota + base)
        return carry

    lax.fori_loop(0, NX // LANES, scat, 0)

    def node_vec(v, carry):
        base = wid * NODES_PER_W + v * LANES
        r = vr[pl.ds(base, LANES)] + PAD
        xi = vx[pl.ds(base, LANES)]
        rowi = iota + v * LANES
        lp = jnp.ones((LANES,), jnp.int32)
        rp = jnp.ones((LANES,), jnp.int32)
        for step in range(N_NEIGH):
            gl = r - lp
            gr = r + rp
            xl = plsc.load_gather(sx, [gl])
            il = plsc.load_gather(si, [gl])
            xr_ = plsc.load_gather(sx, [gr])
            ir = plsc.load_gather(si, [gr])
            dl = jnp.abs(xi - xl)
            dr = jnp.abs(xi - xr_)
            take_l = (dl < dr) | ((dl == dr) & (il < ir))
            sel = jnp.where(take_l, il, ir)
            col = jnp.full((LANES,), step, jnp.int32)
            plsc.store_scatter(vout, [rowi, col], sel)
            inc = take_l.astype(jnp.int32)
            lp = lp + inc
            rp = rp + (1 - inc)
        return carry

    lax.fori_loop(0, NODES_PER_W // LANES, node_vec, 0)

    pltpu.sync_copy(vout, out_hbm.at[pl.ds(wid * NODES_PER_W, NODES_PER_W)])


def _knn_sc(x0, ranks):
    mesh = plsc.VectorSubcoreMesh(
        core_axis_name="c", subcore_axis_name="s",
        num_cores=NC, num_subcores=NS,
    )
    fn = pl.kernel(
        _sc_knn_body,
        out_type=jax.ShapeDtypeStruct((NX, N_NEIGH), jnp.int32),
        mesh=mesh,
        compiler_params=pltpu.CompilerParams(needs_layout_passes=False),
        scratch_types=[
            pltpu.VMEM((NX,), jnp.float32),
            pltpu.VMEM((NX,), jnp.int32),
            pltpu.VMEM((NPAD,), jnp.float32),
            pltpu.VMEM((NPAD,), jnp.int32),
            pltpu.VMEM((NODES_PER_W, N_NEIGH), jnp.int32),
        ],
    )
    return fn(x0, ranks)


def kernel(data, labels, x, steps, bc_left, bc_right, c):
    B, tw, nx = data.shape
    x0 = x[0]

    ranks = _ranks_tc(x0.reshape(1, nx))
    src0 = _knn_sc(x0, ranks)

    u = jnp.transpose(data, (0, 2, 1)).reshape(B * nx, tw)
    y = jnp.transpose(labels, (0, 2, 1)).reshape(B * nx, tw)

    x_pos = jnp.broadcast_to(x0[None, :], (B, nx)).reshape(B * nx)
    t = jnp.linspace(TMIN, TMAX, NT)
    t_sel = t[steps]
    t_pos = jnp.broadcast_to(t_sel[:, None], (B, nx)).reshape(B * nx)
    batch = lax.broadcasted_iota(jnp.int32, (B, nx), 0).reshape(B * nx)

    offs = (jnp.arange(B, dtype=jnp.int32) * nx)[:, None]
    src = (src0.reshape(1, nx * N_NEIGH) + offs).reshape(1, B * nx * N_NEIGH)
    dst0 = lax.broadcasted_iota(jnp.int32, (nx, N_NEIGH), 0).reshape(1, nx * N_NEIGH)
    dst = (dst0 + offs).reshape(1, B * nx * N_NEIGH)
    edge_index = jnp.concatenate([src, dst], 0)

    pos = jnp.concatenate([t_pos[:, None], x_pos[:, None]], 1)
    bc_l = jnp.broadcast_to(bc_left[:, None], (B, nx)).reshape(B * nx, 1)
    bc_r = jnp.broadcast_to(bc_right[:, None], (B, nx)).reshape(B * nx, 1)
    c_n = jnp.broadcast_to(c[:, None], (B, nx)).reshape(B * nx, 1)
    return (u, edge_index, y, pos, batch, bc_l, bc_r, c_n)


# speedup vs baseline: 2.0218x; 1.1245x over previous
import functools

import jax
import jax.numpy as jnp
from jax import lax
from jax.experimental import pallas as pl
from jax.experimental.pallas import tpu as pltpu
from jax.experimental.pallas import tpu_sc as plsc

TW = 5
N_NEIGH = 16
NT = 250
TMIN = 0.0
TMAX = 1.0

NX = 4096
ROW_TILE = 256
PAD = 16
NPAD = NX + 2 * PAD

B_BATCH = 16
NC = 2
NS = 16
NW = NC * NS
NODES_PER_W = NX // NW
LANES = 16

_BIG_X = 3e37
_BIG_I = 1 << 22


def _rank_body(xr_ref, xc_ref, out_ref):
    i = pl.program_id(0)
    ki = lax.bitcast_convert_type(
        jnp.transpose(xc_ref[...], (1, 0)), jnp.int32)
    kj = lax.bitcast_convert_type(xr_ref[...], jnp.int32)
    jj = lax.broadcasted_iota(jnp.int32, (ROW_TILE, NX), 1)
    ii = lax.broadcasted_iota(jnp.int32, (ROW_TILE, NX), 0) + i * ROW_TILE
    before = (kj - (jj < ii).astype(jnp.int32)) < ki
    out_ref[...] = jnp.sum(before.astype(jnp.int32), axis=1, keepdims=True)


def _ranks_tc(xrow):
    ranks = pl.pallas_call(
        _rank_body,
        grid=(NX // ROW_TILE,),
        in_specs=[
            pl.BlockSpec((1, NX), lambda i: (0, 0)),
            pl.BlockSpec((1, ROW_TILE), lambda i: (0, i)),
        ],
        out_specs=pl.BlockSpec((ROW_TILE, 1), lambda i: (i, 0)),
        out_shape=jax.ShapeDtypeStruct((NX, 1), jnp.int32),
    )(xrow, xrow)
    return ranks.reshape(NX)


def _sc_knn_body(x_hbm, r_hbm, out_hbm, vx, vr, sx, si, vout, vsrc_all, vdst_all):
    cid = lax.axis_index("c")
    sid = lax.axis_index("s")
    wid = cid * NS + sid
    epw = NODES_PER_W * N_NEIGH

    pltpu.sync_copy(x_hbm, vx)
    pltpu.sync_copy(r_hbm, vr)

    big_x = jnp.full((LANES,), _BIG_X, jnp.float32)
    big_i = jnp.full((LANES,), _BIG_I, jnp.int32)
    sx[pl.ds(0, LANES)] = big_x
    sx[pl.ds(NX + PAD, LANES)] = big_x
    si[pl.ds(0, LANES)] = big_i
    si[pl.ds(NX + PAD, LANES)] = big_i

    iota = lax.iota(jnp.int32, LANES)

    def scat(cidx, carry):
        base = cidx * LANES
        rv = vr[pl.ds(base, LANES)] + PAD
        plsc.store_scatter(sx, [rv], vx[pl.ds(base, LANES)])
        plsc.store_scatter(si, [rv], iota + base)
        return carry

    lax.fori_loop(0, NX // LANES, scat, 0)

    def node_vec(v, carry):
        base = wid * NODES_PER_W + v * LANES
        r = vr[pl.ds(base, LANES)] + PAD
        xi = vx[pl.ds(base, LANES)]
        flat0 = (iota + v * LANES) * N_NEIGH
        lp = jnp.ones((LANES,), jnp.int32)
        rp = jnp.ones((LANES,), jnp.int32)
        for step in range(N_NEIGH):
            gl = r - lp
            gr = r + rp
            xl = plsc.load_gather(sx, [gl])
            il = plsc.load_gather(si, [gl])
            xr_ = plsc.load_gather(sx, [gr])
            ir = plsc.load_gather(si, [gr])
            dl = jnp.abs(xi - xl)
            dr = jnp.abs(xi - xr_)
            take_l = (dl < dr) | ((dl == dr) & (il < ir))
            sel = jnp.where(take_l, il, ir)
            plsc.store_scatter(vout, [flat0 + step], sel)
            inc = take_l.astype(jnp.int32)
            lp = lp + inc
            rp = rp + (1 - inc)
        return carry

    lax.fori_loop(0, NODES_PER_W // LANES, node_vec, 0)

    def expand_b(b, carry):
        boff = b * NX

        def expand_v(cv, carry2):
            sl = pl.ds(cv * LANES, LANES)
            vsrc_all[b, sl] = vout[sl] + boff
            vdst_all[b, sl] = jnp.zeros((LANES,), jnp.int32) + (
                wid * NODES_PER_W + cv + boff)
            return carry2

        lax.fori_loop(0, epw // LANES, expand_v, 0)
        return carry

    lax.fori_loop(0, B_BATCH, expand_b, 0)

    pltpu.sync_copy(vsrc_all, out_hbm.at[0, :, pl.ds(wid * epw, epw)])
    pltpu.sync_copy(vdst_all, out_hbm.at[1, :, pl.ds(wid * epw, epw)])


def _knn_sc(x0, ranks):
    mesh = plsc.VectorSubcoreMesh(
        core_axis_name="c", subcore_axis_name="s",
        num_cores=NC, num_subcores=NS,
    )
    epw = NODES_PER_W * N_NEIGH
    fn = pl.kernel(
        _sc_knn_body,
        out_type=jax.ShapeDtypeStruct((2, B_BATCH, NX * N_NEIGH), jnp.int32),
        mesh=mesh,
        compiler_params=pltpu.CompilerParams(needs_layout_passes=False),
        scratch_types=[
            pltpu.VMEM((NX,), jnp.float32),
            pltpu.VMEM((NX,), jnp.int32),
            pltpu.VMEM((NPAD,), jnp.float32),
            pltpu.VMEM((NPAD,), jnp.int32),
            pltpu.VMEM((epw,), jnp.int32),
            pltpu.VMEM((B_BATCH, epw), jnp.int32),
            pltpu.VMEM((B_BATCH, epw), jnp.int32),
        ],
    )
    return fn(x0, ranks)


def kernel(data, labels, x, steps, bc_left, bc_right, c):
    B, tw, nx = data.shape
    x0 = x[0]

    ranks = _ranks_tc(x0.reshape(1, nx))
    edge3 = _knn_sc(x0, ranks)

    u = jnp.transpose(data, (0, 2, 1)).reshape(B * nx, tw)
    y = jnp.transpose(labels, (0, 2, 1)).reshape(B * nx, tw)

    x_pos = jnp.broadcast_to(x0[None, :], (B, nx)).reshape(B * nx)
    t = jnp.linspace(TMIN, TMAX, NT)
    t_sel = t[steps]
    t_pos = jnp.broadcast_to(t_sel[:, None], (B, nx)).reshape(B * nx)
    batch = lax.broadcasted_iota(jnp.int32, (B, nx), 0).reshape(B * nx)

    edge_index = edge3.reshape(2, B * nx * N_NEIGH)

    pos = jnp.concatenate([t_pos[:, None], x_pos[:, None]], 1)
    bc_l = jnp.broadcast_to(bc_left[:, None], (B, nx)).reshape(B * nx, 1)
    bc_r = jnp.broadcast_to(bc_right[:, None], (B, nx)).reshape(B * nx, 1)
    c_n = jnp.broadcast_to(c[:, None], (B, nx)).reshape(B * nx, 1)
    return (u, edge_index, y, pos, batch, bc_l, bc_r, c_n)


# speedup vs baseline: 2.0394x; 1.0087x over previous
import functools

import jax
import jax.numpy as jnp
from jax import lax
from jax.experimental import pallas as pl
from jax.experimental.pallas import tpu as pltpu
from jax.experimental.pallas import tpu_sc as plsc

TW = 5
N_NEIGH = 16
NT = 250
TMIN = 0.0
TMAX = 1.0

NX = 4096
ROW_TILE = 256
PAD = 16
NPAD = NX + 2 * PAD

B_BATCH = 16
NC = 2
NS = 16
NW = NC * NS
NODES_PER_W = NX // NW
LANES = 16

_BIG_X = 3e37
_BIG_I = 1 << 22


def _rank_body(xr_ref, xc_ref, out_ref):
    i = pl.program_id(0)
    ki = lax.bitcast_convert_type(xc_ref[...], jnp.int32).reshape(ROW_TILE, 1)
    kj = lax.bitcast_convert_type(xr_ref[...], jnp.int32).reshape(1, NX)
    jj = lax.broadcasted_iota(jnp.int32, (ROW_TILE, NX), 1)
    ii = lax.broadcasted_iota(jnp.int32, (ROW_TILE, NX), 0) + i * ROW_TILE
    before = (kj - (jj < ii).astype(jnp.int32)) < ki
    out_ref[...] = jnp.sum(before.astype(jnp.int32), axis=1)


def _ranks_tc(x0):
    return pl.pallas_call(
        _rank_body,
        grid=(NX // ROW_TILE,),
        in_specs=[
            pl.BlockSpec((NX,), lambda i: (0,)),
            pl.BlockSpec((ROW_TILE,), lambda i: (i,)),
        ],
        out_specs=pl.BlockSpec((ROW_TILE,), lambda i: (i,)),
        out_shape=jax.ShapeDtypeStruct((NX,), jnp.int32),
    )(x0, x0)


def _sc_knn_body(x_hbm, r_hbm, out_hbm, vx, vr, sx, si, vout, vsrc_all, vdst_all):
    cid = lax.axis_index("c")
    sid = lax.axis_index("s")
    wid = cid * NS + sid
    epw = NODES_PER_W * N_NEIGH

    pltpu.sync_copy(x_hbm, vx)
    pltpu.sync_copy(r_hbm, vr)

    big_x = jnp.full((LANES,), _BIG_X, jnp.float32)
    big_i = jnp.full((LANES,), _BIG_I, jnp.int32)
    sx[pl.ds(0, LANES)] = big_x
    sx[pl.ds(NX + PAD, LANES)] = big_x
    si[pl.ds(0, LANES)] = big_i
    si[pl.ds(NX + PAD, LANES)] = big_i

    iota = lax.iota(jnp.int32, LANES)

    def scat(cidx, carry):
        base = cidx * LANES
        rv = vr[pl.ds(base, LANES)] + PAD
        plsc.store_scatter(sx, [rv], vx[pl.ds(base, LANES)])
        plsc.store_scatter(si, [rv], iota + base)
        return carry

    lax.fori_loop(0, NX // LANES, scat, 0)

    def node_vec(v, carry):
        base = wid * NODES_PER_W + v * LANES
        r = vr[pl.ds(base, LANES)] + PAD
        xi = vx[pl.ds(base, LANES)]
        flat0 = (iota + v * LANES) * N_NEIGH
        lp = jnp.ones((LANES,), jnp.int32)
        rp = jnp.ones((LANES,), jnp.int32)
        for step in range(N_NEIGH):
            gl = r - lp
            gr = r + rp
            xl = plsc.load_gather(sx, [gl])
            il = plsc.load_gather(si, [gl])
            xr_ = plsc.load_gather(sx, [gr])
            ir = plsc.load_gather(si, [gr])
            dl = jnp.abs(xi - xl)
            dr = jnp.abs(xi - xr_)
            take_l = (dl < dr) | ((dl == dr) & (il < ir))
            sel = jnp.where(take_l, il, ir)
            plsc.store_scatter(vout, [flat0 + step], sel)
            inc = take_l.astype(jnp.int32)
            lp = lp + inc
            rp = rp + (1 - inc)
        return carry

    lax.fori_loop(0, NODES_PER_W // LANES, node_vec, 0)

    def expand_b(b, carry):
        boff = b * NX

        def expand_v(cv, carry2):
            sl = pl.ds(cv * LANES, LANES)
            vsrc_all[b, sl] = vout[sl] + boff
            vdst_all[b, sl] = jnp.zeros((LANES,), jnp.int32) + (
                wid * NODES_PER_W + cv + boff)
            return carry2

        lax.fori_loop(0, epw // LANES, expand_v, 0)
        return carry

    lax.fori_loop(0, B_BATCH, expand_b, 0)

    pltpu.sync_copy(vsrc_all, out_hbm.at[0, :, pl.ds(wid * epw, epw)])
    pltpu.sync_copy(vdst_all, out_hbm.at[1, :, pl.ds(wid * epw, epw)])


def _knn_sc(x0, ranks):
    mesh = plsc.VectorSubcoreMesh(
        core_axis_name="c", subcore_axis_name="s",
        num_cores=NC, num_subcores=NS,
    )
    epw = NODES_PER_W * N_NEIGH
    fn = pl.kernel(
        _sc_knn_body,
        out_type=jax.ShapeDtypeStruct((2, B_BATCH, NX * N_NEIGH), jnp.int32),
        mesh=mesh,
        compiler_params=pltpu.CompilerParams(needs_layout_passes=False),
        scratch_types=[
            pltpu.VMEM((NX,), jnp.float32),
            pltpu.VMEM((NX,), jnp.int32),
            pltpu.VMEM((NPAD,), jnp.float32),
            pltpu.VMEM((NPAD,), jnp.int32),
            pltpu.VMEM((epw,), jnp.int32),
            pltpu.VMEM((B_BATCH, epw), jnp.int32),
            pltpu.VMEM((B_BATCH, epw), jnp.int32),
        ],
    )
    return fn(x0, ranks)


def kernel(data, labels, x, steps, bc_left, bc_right, c):
    B, tw, nx = data.shape
    x0 = x[0]

    ranks = _ranks_tc(x0)
    edge3 = _knn_sc(x0, ranks)

    clamp = jnp.float32(3.4e38)
    u = jnp.minimum(jnp.transpose(data, (0, 2, 1)), clamp).reshape(B * nx, tw)
    y = jnp.minimum(jnp.transpose(labels, (0, 2, 1)), clamp).reshape(B * nx, tw)

    x_pos = jnp.broadcast_to(x0[None, :], (B, nx)).reshape(B * nx)
    t = jnp.linspace(TMIN, TMAX, NT)
    t_sel = t[steps]
    t_pos = jnp.broadcast_to(t_sel[:, None], (B, nx)).reshape(B * nx)
    batch = lax.broadcasted_iota(jnp.int32, (B, nx), 0).reshape(B * nx)

    edge_index = edge3.reshape(2, B * nx * N_NEIGH)

    pos = jnp.concatenate([t_pos[:, None], x_pos[:, None]], 1)
    bc_l = jnp.broadcast_to(bc_left[:, None], (B, nx)).reshape(B * nx, 1)
    bc_r = jnp.broadcast_to(bc_right[:, None], (B, nx)).reshape(B * nx, 1)
    c_n = jnp.broadcast_to(c[:, None], (B, nx)).reshape(B * nx, 1)
    return (u, edge_index, y, pos, batch, bc_l, bc_r, c_n)
